# baseline (speedup 1.0000x reference)
import jax
import jax.numpy as jnp
from jax.experimental import pallas as pl

_N = 128
_C = 32768
_BLOCK_ROWS = 64
_CHUNK = 2048


def _ste_block(x_ref, o_ref):
    rows = x_ref.shape[0]
    n_chunks = _C // _CHUNK
    m = jnp.full((rows, 1), -jnp.inf, dtype=jnp.float32)
    idx = jnp.zeros((rows, 1), dtype=jnp.int32)
    for k in range(n_chunks):
        xc = x_ref[:, k * _CHUNK:(k + 1) * _CHUNK]
        lmax = jnp.max(xc, axis=1, keepdims=True)
        larg = jnp.argmax(xc, axis=1).astype(jnp.int32)[:, None] + k * _CHUNK
        take = lmax > m
        m = jnp.where(take, lmax, m)
        idx = jnp.where(take, larg, idx)
    for k in range(n_chunks):
        ii = jax.lax.broadcasted_iota(jnp.int32, (rows, _CHUNK), 1) + k * _CHUNK
        o_ref[:, k * _CHUNK:(k + 1) * _CHUNK] = (ii == idx).astype(jnp.float32)


@jax.jit
def kernel(x):
    grid = (_N // _BLOCK_ROWS,)
    return pl.pallas_call(
        _ste_block,
        grid=grid,
        in_specs=[pl.BlockSpec((_BLOCK_ROWS, _C), lambda i: (i, 0))],
        out_specs=pl.BlockSpec((_BLOCK_ROWS, _C), lambda i: (i, 0)),
        out_shape=jax.ShapeDtypeStruct((_N, _C), jnp.float32),
    )(x)
